# baseline (speedup 1.0000x reference)
import functools

import numpy as np

import jax
import jax.numpy as jnp
from jax.experimental import pallas as pl
from jax.experimental.pallas import tpu as pltpu


def _conv_compute(a, w_ref, b_ref, th, G):
    a = a.reshape(th * G, a.shape[-1])
    acc = jnp.dot(a, w_ref[...], preferred_element_type=jnp.float32)
    acc = jnp.maximum(acc + b_ref[...], 0.0)
    half = acc.shape[-1] // 2
    y = jnp.maximum(acc[:, :half], acc[:, half:])
    return y.reshape(th // 2, 2, G, half).max(axis=1)


def _store_packed(o_ref, i, y, th2):
    c2 = o_ref.shape[-1]
    g2 = o_ref.shape[-2] - 2
    y = y.reshape(th2, g2, c2).astype(o_ref.dtype)
    o_ref[i, 1:th2 + 1, 1:g2 + 1, :] = y
    zrow = jnp.zeros((1, g2 + 2, c2), o_ref.dtype)
    o_ref[i, 0:1] = zrow
    o_ref[i, th2 + 1:th2 + 2] = zrow
    zcol = jnp.zeros((th2 + 2, 1, c2), o_ref.dtype)
    o_ref[i, :, 0:1, :] = zcol
    o_ref[i, :, g2 + 1:g2 + 2, :] = zcol


def _conv_kernel(x_ref, w_ref, b_ref, o_ref, *, B, th, G, P, O, nj, final,
                 nchw=False):
    for i in range(B):
        if nchw:
            C = x_ref.shape[1]
            pieces = [x_ref[i, c, dh:dh + th, j:j + G, :]
                      for dh in (0, 1, 2) for j in range(nj)
                      for c in range(C)]
        else:
            pieces = [x_ref[i, dh:dh + th, j:j + G, :]
                      for dh in (0, 1, 2) for j in range(nj)]
        a = jnp.concatenate(pieces, axis=-1)
        y = _conv_compute(a, w_ref, b_ref, th, G)
        th2 = th // 2
        if final:
            y = y.reshape(th2 * G, y.shape[-1]).astype(o_ref.dtype)
            o_ref[i] = jnp.transpose(y, (1, 0)).reshape(o_ref.shape[1:])
        else:
            _store_packed(o_ref, i, y, th2)


def _pack_w(w_oihw, P, lane_cq, nj):
    wt = jnp.transpose(w_oihw, (2, 3, 1, 0)).astype(jnp.bfloat16)
    order = list(range(0, P, 2)) + list(range(1, P, 2))
    sel = np.zeros((nj, P, P, 3), np.float32)
    for j in range(nj):
        for q in range(P):
            for slot, p in enumerate(order):
                dw = j * P + q - p if nj == 2 else (j - 1) * P + q - p + 1
                if 0 <= dw < 3:
                    sel[j, q, slot, dw] = 1.0
    out_order = 'hjcqpo' if lane_cq else 'hjqcpo'
    wp = jnp.einsum('jqpd,hdco->' + out_order,
                    jnp.asarray(sel, jnp.bfloat16), wt)
    rows = 3 * nj * P * w_oihw.shape[1]
    return wp.reshape(rows, P * w_oihw.shape[0])


_CPARAMS = pltpu.CompilerParams(
    dimension_semantics=("parallel",),
    vmem_limit_bytes=100 * 1024 * 1024)


def _conv_pool(xp, w_oihw, b, P, B, nj=3, final=False, lane_cq=False,
               nchw=False):
    if nchw:
        n, c_in, h2, gtot, _ = xp.shape
        pc = P * c_in
    else:
        n, h2, gtot, pc = xp.shape
    h, G = h2 - 2, gtot - (nj - 1)
    O = w_oihw.shape[0]
    wp = _pack_w(w_oihw, P, lane_cq=lane_cq, nj=nj)
    bp = jnp.tile(b.astype(jnp.float32), P).reshape(1, P * O)
    th2 = h // 2
    w2 = G * P // 2
    if final:
        out_sd = jax.ShapeDtypeStruct((n, O, th2 * w2), jnp.bfloat16)
        out_block = (B, O, th2 * w2)
    else:
        c2 = 128
        out_sd = jax.ShapeDtypeStruct((n, th2 + 2, w2 * O // c2 + 2, c2),
                                      jnp.bfloat16)
        out_block = (B, th2 + 2, w2 * O // c2 + 2, c2)
    body = functools.partial(_conv_kernel, B=B, th=h, G=G, P=P, O=O, nj=nj,
                             final=final, nchw=nchw)
    if nchw:
        in_block = (B, c_in, h + 2, gtot, P)
    else:
        in_block = (B, h + 2, gtot, pc)
    return pl.pallas_call(
        body,
        out_shape=out_sd,
        grid=(n // B,),
        in_specs=[
            pl.BlockSpec(in_block,
                         lambda bi: (bi,) + (0,) * (len(in_block) - 1)),
            pl.BlockSpec((3 * nj * pc, P * O), lambda bi: (0, 0)),
            pl.BlockSpec((1, P * O), lambda bi: (0, 0)),
        ],
        out_specs=pl.BlockSpec(out_block,
                               lambda bi: (bi,) + (0,) * (len(out_block) - 1)),
        compiler_params=_CPARAMS,
    )(xp, wp, bp)


def _fc1_kernel(x_ref, w1_ref, o_ref, acc_ref):
    k = pl.program_id(1)

    @pl.when(k == 0)
    def _init():
        acc_ref[...] = jnp.zeros_like(acc_ref)

    w = w1_ref[...].astype(jnp.bfloat16)
    acc_ref[...] += jax.lax.dot_general(
        x_ref[...], w, (((1,), (1,)), ((), ())),
        preferred_element_type=jnp.float32)

    @pl.when(k == pl.num_programs(1) - 1)
    def _finalize():
        o_ref[0] = acc_ref[...]


def _fc2_kernel(p_ref, b1_ref, w2_ref, b2_ref, o_ref):
    h = p_ref[0] + p_ref[1] + b1_ref[...]
    h = jnp.maximum(h, 0.0).astype(jnp.bfloat16)
    y = jax.lax.dot_general(
        h, w2_ref[...].astype(jnp.bfloat16), (((1,), (1,)), ((), ())),
        preferred_element_type=jnp.float32)
    o_ref[...] = y + b2_ref[...]


def _fc_tail(x, w1, b1, w2, b2, *, tk=2048):
    m, k = x.shape
    n1 = w1.shape[0]
    n2 = w2.shape[0]
    kt = k // tk
    part = pl.pallas_call(
        _fc1_kernel,
        out_shape=jax.ShapeDtypeStruct((2, m, n1), jnp.float32),
        grid=(2, kt // 2),
        in_specs=[
            pl.BlockSpec((m, tk), lambda c, kk: (0, c * (kt // 2) + kk)),
            pl.BlockSpec((n1, tk), lambda c, kk: (0, c * (kt // 2) + kk)),
        ],
        out_specs=pl.BlockSpec((1, m, n1), lambda c, kk: (c, 0, 0)),
        scratch_shapes=[pltpu.VMEM((m, n1), jnp.float32)],
        compiler_params=pltpu.CompilerParams(
            dimension_semantics=("parallel", "arbitrary"),
            vmem_limit_bytes=100 * 1024 * 1024),
    )(x, w1)
    b1p = b1.astype(jnp.float32).reshape(1, n1)
    b2p = b2.astype(jnp.float32).reshape(1, n2)
    return pl.pallas_call(
        _fc2_kernel,
        out_shape=jax.ShapeDtypeStruct((m, n2), jnp.float32),
        in_specs=[
            pl.BlockSpec((2, m, n1), lambda: (0, 0, 0)),
            pl.BlockSpec((1, n1), lambda: (0, 0)),
            pl.BlockSpec((n2, n1), lambda: (0, 0)),
            pl.BlockSpec((1, n2), lambda: (0, 0)),
        ],
        out_specs=pl.BlockSpec((m, n2), lambda: (0, 0)),
        compiler_params=pltpu.CompilerParams(
            vmem_limit_bytes=100 * 1024 * 1024),
    )(part, b1p, w2, b2p)


def kernel(conv1_w, conv1_b, conv2_w, conv2_b, conv3_w, conv3_b,
           fc1_w, fc1_b, fc2_w, fc2_b, x_nchw):
    n, c, h, w = x_nchw.shape
    x = jnp.pad(x_nchw.astype(jnp.bfloat16),
                ((0, 0), (0, 0), (1, 1), (1, 31)))
    x = x.reshape(n, c, h + 2, 5, 32)
    y = _conv_pool(x, conv1_w, conv1_b, P=32, B=8, nj=2, lane_cq=True,
                   nchw=True)
    y = _conv_pool(y, conv2_w, conv2_b, P=4, B=8)
    y = _conv_pool(y, conv3_w, conv3_b, P=2, B=8, final=True)
    x = y.reshape(n, -1)
    return _fc_tail(x, fc1_w, fc1_b, fc2_w, fc2_b)


# speedup vs baseline: 1.5665x; 1.5665x over previous
import functools

import numpy as np

import jax
import jax.numpy as jnp
from jax.experimental import pallas as pl
from jax.experimental.pallas import tpu as pltpu


def _conv_compute(a, w_ref, b_ref, th, G):
    a = a.reshape(th * G, a.shape[-1])
    acc = jnp.dot(a, w_ref[...], preferred_element_type=jnp.float32)
    acc = jnp.maximum(acc + b_ref[...], 0.0)
    half = acc.shape[-1] // 2
    y = jnp.maximum(acc[:, :half], acc[:, half:])
    return y.reshape(th // 2, 2, G, half).max(axis=1)


def _store_packed(o_ref, i, y, th2):
    c2 = o_ref.shape[-1]
    g2 = o_ref.shape[-2] - 2
    y = y.reshape(th2, g2, c2).astype(o_ref.dtype)
    o_ref[i, 1:th2 + 1, 1:g2 + 1, :] = y
    zrow = jnp.zeros((1, g2 + 2, c2), o_ref.dtype)
    o_ref[i, 0:1] = zrow
    o_ref[i, th2 + 1:th2 + 2] = zrow
    zcol = jnp.zeros((th2 + 2, 1, c2), o_ref.dtype)
    o_ref[i, :, 0:1, :] = zcol
    o_ref[i, :, g2 + 1:g2 + 2, :] = zcol


def _conv_kernel(x_ref, w_ref, b_ref, o_ref, *, B, th, G, P, O, nj, final,
                 nchw=False):
    for i in range(B):
        if nchw:
            C = x_ref.shape[1]
            pieces = [x_ref[i, c, dh:dh + th, j:j + G, :]
                      for dh in (0, 1, 2) for j in range(nj)
                      for c in range(C)]
        else:
            pieces = [x_ref[i, dh:dh + th, j:j + G, :]
                      for dh in (0, 1, 2) for j in range(nj)]
        a = jnp.concatenate(pieces, axis=-1)
        y = _conv_compute(a, w_ref, b_ref, th, G)
        th2 = th // 2
        if final:
            y = y.reshape(th2 * G, y.shape[-1]).astype(o_ref.dtype)
            o_ref[i] = jnp.transpose(y, (1, 0)).reshape(o_ref.shape[1:])
        else:
            _store_packed(o_ref, i, y, th2)


def _pack_w(w_oihw, P, lane_cq, nj):
    wt = jnp.transpose(w_oihw, (2, 3, 1, 0)).astype(jnp.bfloat16)
    order = list(range(0, P, 2)) + list(range(1, P, 2))
    sel = np.zeros((nj, P, P, 3), np.float32)
    for j in range(nj):
        for q in range(P):
            for slot, p in enumerate(order):
                dw = j * P + q - p if nj == 2 else (j - 1) * P + q - p + 1
                if 0 <= dw < 3:
                    sel[j, q, slot, dw] = 1.0
    out_order = 'hjcqpo' if lane_cq else 'hjqcpo'
    wp = jnp.einsum('jqpd,hdco->' + out_order,
                    jnp.asarray(sel, jnp.bfloat16), wt)
    rows = 3 * nj * P * w_oihw.shape[1]
    return wp.reshape(rows, P * w_oihw.shape[0])


_CPARAMS = pltpu.CompilerParams(
    dimension_semantics=("parallel",),
    vmem_limit_bytes=100 * 1024 * 1024)


def _conv_pool(xp, w_oihw, b, P, B, nj=3, final=False, lane_cq=False,
               nchw=False):
    if nchw:
        n, c_in, h2, gtot, _ = xp.shape
        pc = P * c_in
    else:
        n, h2, gtot, pc = xp.shape
    h, G = h2 - 2, gtot - (nj - 1)
    O = w_oihw.shape[0]
    wp = _pack_w(w_oihw, P, lane_cq=lane_cq, nj=nj)
    bp = jnp.tile(b.astype(jnp.float32), P).reshape(1, P * O)
    th2 = h // 2
    w2 = G * P // 2
    if final:
        out_sd = jax.ShapeDtypeStruct((n, O, th2 * w2), jnp.bfloat16)
        out_block = (B, O, th2 * w2)
    else:
        c2 = 128
        out_sd = jax.ShapeDtypeStruct((n, th2 + 2, w2 * O // c2 + 2, c2),
                                      jnp.bfloat16)
        out_block = (B, th2 + 2, w2 * O // c2 + 2, c2)
    body = functools.partial(_conv_kernel, B=B, th=h, G=G, P=P, O=O, nj=nj,
                             final=final, nchw=nchw)
    if nchw:
        in_block = (B, c_in, h + 2, gtot, P)
    else:
        in_block = (B, h + 2, gtot, pc)
    return pl.pallas_call(
        body,
        out_shape=out_sd,
        grid=(n // B,),
        in_specs=[
            pl.BlockSpec(in_block,
                         lambda bi: (bi,) + (0,) * (len(in_block) - 1)),
            pl.BlockSpec((3 * nj * pc, P * O), lambda bi: (0, 0)),
            pl.BlockSpec((1, P * O), lambda bi: (0, 0)),
        ],
        out_specs=pl.BlockSpec(out_block,
                               lambda bi: (bi,) + (0,) * (len(out_block) - 1)),
        compiler_params=_CPARAMS,
    )(xp, wp, bp)


def _fc1_kernel(x_ref, w1_ref, o_ref, acc_ref):
    k = pl.program_id(1)

    @pl.when(k == 0)
    def _init():
        acc_ref[...] = jnp.zeros_like(acc_ref)

    w = w1_ref[...].astype(jnp.bfloat16)
    acc_ref[...] += jax.lax.dot_general(
        x_ref[...], w, (((1,), (1,)), ((), ())),
        preferred_element_type=jnp.float32)

    @pl.when(k == pl.num_programs(1) - 1)
    def _finalize():
        o_ref[0] = acc_ref[...]


def _fc2_kernel(p_ref, b1_ref, w2_ref, b2_ref, o_ref):
    h = p_ref[0] + p_ref[1] + b1_ref[...]
    h = jnp.maximum(h, 0.0).astype(jnp.bfloat16)
    y = jax.lax.dot_general(
        h, w2_ref[...].astype(jnp.bfloat16), (((1,), (1,)), ((), ())),
        preferred_element_type=jnp.float32)
    o_ref[...] = y + b2_ref[...]


def _fc_tail(x, w1, b1, w2, b2, *, tk=2048):
    m, k = x.shape
    n1 = w1.shape[0]
    n2 = w2.shape[0]
    kt = k // tk
    part = pl.pallas_call(
        _fc1_kernel,
        out_shape=jax.ShapeDtypeStruct((2, m, n1), jnp.float32),
        grid=(2, kt // 2),
        in_specs=[
            pl.BlockSpec((m, tk), lambda c, kk: (0, c * (kt // 2) + kk)),
            pl.BlockSpec((n1, tk), lambda c, kk: (0, c * (kt // 2) + kk)),
        ],
        out_specs=pl.BlockSpec((1, m, n1), lambda c, kk: (c, 0, 0)),
        scratch_shapes=[pltpu.VMEM((m, n1), jnp.float32)],
        compiler_params=pltpu.CompilerParams(
            dimension_semantics=("parallel", "arbitrary"),
            vmem_limit_bytes=100 * 1024 * 1024),
    )(x, w1)
    b1p = b1.astype(jnp.float32).reshape(1, n1)
    b2p = b2.astype(jnp.float32).reshape(1, n2)
    return pl.pallas_call(
        _fc2_kernel,
        out_shape=jax.ShapeDtypeStruct((m, n2), jnp.float32),
        in_specs=[
            pl.BlockSpec((2, m, n1), lambda: (0, 0, 0)),
            pl.BlockSpec((1, n1), lambda: (0, 0)),
            pl.BlockSpec((n2, n1), lambda: (0, 0)),
            pl.BlockSpec((1, n2), lambda: (0, 0)),
        ],
        out_specs=pl.BlockSpec((m, n2), lambda: (0, 0)),
        compiler_params=pltpu.CompilerParams(
            vmem_limit_bytes=100 * 1024 * 1024),
    )(part, b1p, w2, b2p)


def kernel(conv1_w, conv1_b, conv2_w, conv2_b, conv3_w, conv3_b,
           fc1_w, fc1_b, fc2_w, fc2_b, x_nchw):
    n, c, h, w = x_nchw.shape
    x = jnp.pad(x_nchw.astype(jnp.bfloat16),
                ((0, 0), (0, 0), (1, 1), (1, 31)))
    x = x.reshape(n, c, h + 2, 5, 32)
    x = jnp.transpose(x, (0, 2, 3, 1, 4)).reshape(n, h + 2, 5, 128)
    y = _conv_pool(x, conv1_w, conv1_b, P=32, B=8, nj=2, lane_cq=True)
    y = _conv_pool(y, conv2_w, conv2_b, P=4, B=8)
    y = _conv_pool(y, conv3_w, conv3_b, P=2, B=8, final=True)
    x = y.reshape(n, -1)
    return _fc_tail(x, fc1_w, fc1_b, fc2_w, fc2_b)


# speedup vs baseline: 1.5772x; 1.0068x over previous
import functools

import numpy as np

import jax
import jax.numpy as jnp
from jax.experimental import pallas as pl
from jax.experimental.pallas import tpu as pltpu


def _conv_compute(a, w_ref, b_ref, th, G):
    a = a.reshape(th * G, a.shape[-1])
    acc = jnp.dot(a, w_ref[...], preferred_element_type=jnp.float32)
    acc = jnp.maximum(acc + b_ref[...], 0.0)
    half = acc.shape[-1] // 2
    y = jnp.maximum(acc[:, :half], acc[:, half:])
    return y.reshape(th // 2, 2, G, half).max(axis=1)


def _store_packed(o_ref, i, y, th2):
    c2 = o_ref.shape[-1]
    g2 = o_ref.shape[-2] - 2
    y = y.reshape(th2, g2, c2).astype(o_ref.dtype)
    o_ref[i, 1:th2 + 1, 1:g2 + 1, :] = y
    zrow = jnp.zeros((1, g2 + 2, c2), o_ref.dtype)
    o_ref[i, 0:1] = zrow
    o_ref[i, th2 + 1:th2 + 2] = zrow
    zcol = jnp.zeros((th2 + 2, 1, c2), o_ref.dtype)
    o_ref[i, :, 0:1, :] = zcol
    o_ref[i, :, g2 + 1:g2 + 2, :] = zcol


def _conv_kernel(x_ref, w_ref, b_ref, o_ref, *, B, th, G, P, O, nj, final):
    for i in range(B):
        pieces = [x_ref[i, dh:dh + th, j:j + G, :]
                  for dh in (0, 1, 2) for j in range(nj)]
        a = jnp.concatenate(pieces, axis=-1)
        y = _conv_compute(a, w_ref, b_ref, th, G)
        th2 = th // 2
        if final:
            y = y.reshape(th2 * G, y.shape[-1]).astype(o_ref.dtype)
            o_ref[i] = jnp.transpose(y, (1, 0)).reshape(o_ref.shape[1:])
        else:
            _store_packed(o_ref, i, y, th2)


def _pack_w(w_oihw, P, lane_cq, nj):
    wt = jnp.transpose(w_oihw, (2, 3, 1, 0)).astype(jnp.bfloat16)
    order = list(range(0, P, 2)) + list(range(1, P, 2))
    sel = np.zeros((nj, P, P, 3), np.float32)
    for j in range(nj):
        for q in range(P):
            for slot, p in enumerate(order):
                dw = j * P + q - p if nj == 2 else (j - 1) * P + q - p + 1
                if 0 <= dw < 3:
                    sel[j, q, slot, dw] = 1.0
    out_order = 'hjcqpo' if lane_cq else 'hjqcpo'
    wp = jnp.einsum('jqpd,hdco->' + out_order,
                    jnp.asarray(sel, jnp.bfloat16), wt)
    rows = 3 * nj * P * w_oihw.shape[1]
    return wp.reshape(rows, P * w_oihw.shape[0])


_CPARAMS = pltpu.CompilerParams(
    dimension_semantics=("parallel",),
    vmem_limit_bytes=100 * 1024 * 1024)


def _conv_pool(xp, w_oihw, b, P, B, nj=3, final=False, lane_cq=False):
    n, h2, gtot, pc = xp.shape
    h, G = h2 - 2, gtot - (nj - 1)
    O = w_oihw.shape[0]
    wp = _pack_w(w_oihw, P, lane_cq=lane_cq, nj=nj)
    bp = jnp.tile(b.astype(jnp.float32), P).reshape(1, P * O)
    th2 = h // 2
    w2 = G * P // 2
    if final:
        out_sd = jax.ShapeDtypeStruct((n, O, th2 * w2), jnp.bfloat16)
        out_block = (B, O, th2 * w2)
    else:
        c2 = 128
        out_sd = jax.ShapeDtypeStruct((n, th2 + 2, w2 * O // c2 + 2, c2),
                                      jnp.bfloat16)
        out_block = (B, th2 + 2, w2 * O // c2 + 2, c2)
    body = functools.partial(_conv_kernel, B=B, th=h, G=G, P=P, O=O, nj=nj,
                             final=final)
    return pl.pallas_call(
        body,
        out_shape=out_sd,
        grid=(n // B,),
        in_specs=[
            pl.BlockSpec((B, h + 2, gtot, pc), lambda bi: (bi, 0, 0, 0)),
            pl.BlockSpec((3 * nj * pc, P * O), lambda bi: (0, 0)),
            pl.BlockSpec((1, P * O), lambda bi: (0, 0)),
        ],
        out_specs=pl.BlockSpec(out_block,
                               lambda bi: (bi,) + (0,) * (len(out_block) - 1)),
        compiler_params=_CPARAMS,
    )(xp, wp, bp)


def _fc1_kernel(x_ref, w1_ref, o_ref, acc_ref):
    k = pl.program_id(1)

    @pl.when(k == 0)
    def _init():
        acc_ref[...] = jnp.zeros_like(acc_ref)

    w = w1_ref[...].astype(jnp.bfloat16)
    acc_ref[...] += jax.lax.dot_general(
        x_ref[...], w, (((1,), (1,)), ((), ())),
        preferred_element_type=jnp.float32)

    @pl.when(k == pl.num_programs(1) - 1)
    def _finalize():
        o_ref[0] = acc_ref[...]


def _fc2_kernel(p_ref, b1_ref, w2_ref, b2_ref, o_ref):
    h = p_ref[0] + p_ref[1] + b1_ref[...]
    h = jnp.maximum(h, 0.0).astype(jnp.bfloat16)
    y = jax.lax.dot_general(
        h, w2_ref[...].astype(jnp.bfloat16), (((1,), (1,)), ((), ())),
        preferred_element_type=jnp.float32)
    o_ref[...] = y + b2_ref[...]


def _fc_tail(x, w1, b1, w2, b2, *, tk=2048):
    m, k = x.shape
    n1 = w1.shape[0]
    n2 = w2.shape[0]
    kt = k // tk
    part = pl.pallas_call(
        _fc1_kernel,
        out_shape=jax.ShapeDtypeStruct((2, m, n1), jnp.float32),
        grid=(2, kt // 2),
        in_specs=[
            pl.BlockSpec((m, tk), lambda c, kk: (0, c * (kt // 2) + kk)),
            pl.BlockSpec((n1, tk), lambda c, kk: (0, c * (kt // 2) + kk)),
        ],
        out_specs=pl.BlockSpec((1, m, n1), lambda c, kk: (c, 0, 0)),
        scratch_shapes=[pltpu.VMEM((m, n1), jnp.float32)],
        compiler_params=pltpu.CompilerParams(
            dimension_semantics=("parallel", "arbitrary"),
            vmem_limit_bytes=100 * 1024 * 1024),
    )(x, w1)
    b1p = b1.astype(jnp.float32).reshape(1, n1)
    b2p = b2.astype(jnp.float32).reshape(1, n2)
    return pl.pallas_call(
        _fc2_kernel,
        out_shape=jax.ShapeDtypeStruct((m, n2), jnp.float32),
        in_specs=[
            pl.BlockSpec((2, m, n1), lambda: (0, 0, 0)),
            pl.BlockSpec((1, n1), lambda: (0, 0)),
            pl.BlockSpec((n2, n1), lambda: (0, 0)),
            pl.BlockSpec((1, n2), lambda: (0, 0)),
        ],
        out_specs=pl.BlockSpec((m, n2), lambda: (0, 0)),
        compiler_params=pltpu.CompilerParams(
            vmem_limit_bytes=100 * 1024 * 1024),
    )(part, b1p, w2, b2p)


def kernel(conv1_w, conv1_b, conv2_w, conv2_b, conv3_w, conv3_b,
           fc1_w, fc1_b, fc2_w, fc2_b, x_nchw):
    n, c, h, w = x_nchw.shape
    x = jnp.pad(x_nchw.astype(jnp.bfloat16),
                ((0, 0), (0, 0), (1, 1), (1, 31)))
    x = x.reshape(n, c, h + 2, 5, 32)
    x = jnp.transpose(x, (0, 2, 3, 1, 4)).reshape(n, h + 2, 5, 128)
    y = _conv_pool(x, conv1_w, conv1_b, P=32, B=4, nj=2, lane_cq=True)
    y = _conv_pool(y, conv2_w, conv2_b, P=4, B=4)
    y = _conv_pool(y, conv3_w, conv3_b, P=2, B=4, final=True)
    x = y.reshape(n, -1)
    return _fc_tail(x, fc1_w, fc1_b, fc2_w, fc2_b)


# speedup vs baseline: 1.6350x; 1.0366x over previous
import functools

import numpy as np

import jax
import jax.numpy as jnp
from jax.experimental import pallas as pl
from jax.experimental.pallas import tpu as pltpu


def _conv_compute(a, w_ref, b_ref, th, G):
    a = a.reshape(th * G, a.shape[-1])
    acc = jnp.dot(a, w_ref[...], preferred_element_type=jnp.float32)
    acc = jnp.maximum(acc + b_ref[...], 0.0)
    half = acc.shape[-1] // 2
    y = jnp.maximum(acc[:, :half], acc[:, half:])
    return y.reshape(th // 2, 2, G, half).max(axis=1)


def _store_packed(o_ref, i, y, th2):
    c2 = o_ref.shape[-1]
    g2 = o_ref.shape[-2] - 2
    y = y.reshape(th2, g2, c2).astype(o_ref.dtype)
    o_ref[i, 1:th2 + 1, 1:g2 + 1, :] = y
    zrow = jnp.zeros((1, g2 + 2, c2), o_ref.dtype)
    o_ref[i, 0:1] = zrow
    o_ref[i, th2 + 1:th2 + 2] = zrow
    zcol = jnp.zeros((th2 + 2, 1, c2), o_ref.dtype)
    o_ref[i, :, 0:1, :] = zcol
    o_ref[i, :, g2 + 1:g2 + 2, :] = zcol


def _conv_kernel(x_ref, w_ref, b_ref, o_ref, *, B, th, G, P, O, nj, final):
    for i in range(B):
        pieces = [x_ref[i, dh:dh + th, j:j + G, :]
                  for dh in (0, 1, 2) for j in range(nj)]
        a = jnp.concatenate(pieces, axis=-1)
        y = _conv_compute(a, w_ref, b_ref, th, G)
        th2 = th // 2
        if final:
            y = y.reshape(th2 * G, y.shape[-1]).astype(o_ref.dtype)
            o_ref[i] = jnp.transpose(y, (1, 0)).reshape(o_ref.shape[1:])
        else:
            _store_packed(o_ref, i, y, th2)


def _pack_w(w_oihw, P, lane_cq, nj):
    wt = jnp.transpose(w_oihw, (2, 3, 1, 0)).astype(jnp.bfloat16)
    order = list(range(0, P, 2)) + list(range(1, P, 2))
    sel = np.zeros((nj, P, P, 3), np.float32)
    for j in range(nj):
        for q in range(P):
            for slot, p in enumerate(order):
                dw = j * P + q - p if nj == 2 else (j - 1) * P + q - p + 1
                if 0 <= dw < 3:
                    sel[j, q, slot, dw] = 1.0
    out_order = 'hjcqpo' if lane_cq else 'hjqcpo'
    wp = jnp.einsum('jqpd,hdco->' + out_order,
                    jnp.asarray(sel, jnp.bfloat16), wt)
    rows = 3 * nj * P * w_oihw.shape[1]
    return wp.reshape(rows, P * w_oihw.shape[0])


_CPARAMS = pltpu.CompilerParams(
    dimension_semantics=("parallel",),
    vmem_limit_bytes=100 * 1024 * 1024)


def _conv_pool(xp, w_oihw, b, P, B, nj=3, final=False, lane_cq=False):
    n, h2, gtot, pc = xp.shape
    h, G = h2 - 2, gtot - (nj - 1)
    O = w_oihw.shape[0]
    wp = _pack_w(w_oihw, P, lane_cq=lane_cq, nj=nj)
    bp = jnp.tile(b.astype(jnp.float32), P).reshape(1, P * O)
    th2 = h // 2
    w2 = G * P // 2
    if final:
        out_sd = jax.ShapeDtypeStruct((n, O, th2 * w2), jnp.bfloat16)
        out_block = (B, O, th2 * w2)
    else:
        c2 = 128
        out_sd = jax.ShapeDtypeStruct((n, th2 + 2, w2 * O // c2 + 2, c2),
                                      jnp.bfloat16)
        out_block = (B, th2 + 2, w2 * O // c2 + 2, c2)
    body = functools.partial(_conv_kernel, B=B, th=h, G=G, P=P, O=O, nj=nj,
                             final=final)
    return pl.pallas_call(
        body,
        out_shape=out_sd,
        grid=(n // B,),
        in_specs=[
            pl.BlockSpec((B, h + 2, gtot, pc), lambda bi: (bi, 0, 0, 0)),
            pl.BlockSpec((3 * nj * pc, P * O), lambda bi: (0, 0)),
            pl.BlockSpec((1, P * O), lambda bi: (0, 0)),
        ],
        out_specs=pl.BlockSpec(out_block,
                               lambda bi: (bi,) + (0,) * (len(out_block) - 1)),
        compiler_params=_CPARAMS,
    )(xp, wp, bp)


def _fc1_kernel(x_ref, w1_ref, o_ref, acc_ref):
    k = pl.program_id(1)

    @pl.when(k == 0)
    def _init():
        acc_ref[...] = jnp.zeros_like(acc_ref)

    w = w1_ref[...].astype(jnp.bfloat16)
    acc_ref[...] += jax.lax.dot_general(
        x_ref[...], w, (((1,), (1,)), ((), ())),
        preferred_element_type=jnp.float32)

    @pl.when(k == pl.num_programs(1) - 1)
    def _finalize():
        o_ref[0] = acc_ref[...]


def _fc2_kernel(p_ref, b1_ref, w2_ref, b2_ref, o_ref):
    h = p_ref[0] + p_ref[1] + b1_ref[...]
    h = jnp.maximum(h, 0.0).astype(jnp.bfloat16)
    y = jax.lax.dot_general(
        h, w2_ref[...].astype(jnp.bfloat16), (((1,), (1,)), ((), ())),
        preferred_element_type=jnp.float32)
    o_ref[...] = y + b2_ref[...]


def _fc_tail(x, w1, b1, w2, b2, *, tk=4096):
    m, k = x.shape
    n1 = w1.shape[0]
    n2 = w2.shape[0]
    kt = k // tk
    part = pl.pallas_call(
        _fc1_kernel,
        out_shape=jax.ShapeDtypeStruct((2, m, n1), jnp.float32),
        grid=(2, kt // 2),
        in_specs=[
            pl.BlockSpec((m, tk), lambda c, kk: (0, c * (kt // 2) + kk)),
            pl.BlockSpec((n1, tk), lambda c, kk: (0, c * (kt // 2) + kk)),
        ],
        out_specs=pl.BlockSpec((1, m, n1), lambda c, kk: (c, 0, 0)),
        scratch_shapes=[pltpu.VMEM((m, n1), jnp.float32)],
        compiler_params=pltpu.CompilerParams(
            dimension_semantics=("parallel", "arbitrary"),
            vmem_limit_bytes=100 * 1024 * 1024),
    )(x, w1)
    b1p = b1.astype(jnp.float32).reshape(1, n1)
    b2p = b2.astype(jnp.float32).reshape(1, n2)
    return pl.pallas_call(
        _fc2_kernel,
        out_shape=jax.ShapeDtypeStruct((m, n2), jnp.float32),
        in_specs=[
            pl.BlockSpec((2, m, n1), lambda: (0, 0, 0)),
            pl.BlockSpec((1, n1), lambda: (0, 0)),
            pl.BlockSpec((n2, n1), lambda: (0, 0)),
            pl.BlockSpec((1, n2), lambda: (0, 0)),
        ],
        out_specs=pl.BlockSpec((m, n2), lambda: (0, 0)),
        compiler_params=pltpu.CompilerParams(
            vmem_limit_bytes=100 * 1024 * 1024),
    )(part, b1p, w2, b2p)


def kernel(conv1_w, conv1_b, conv2_w, conv2_b, conv3_w, conv3_b,
           fc1_w, fc1_b, fc2_w, fc2_b, x_nchw):
    n, c, h, w = x_nchw.shape
    x = jnp.pad(x_nchw.astype(jnp.bfloat16),
                ((0, 0), (0, 0), (1, 1), (1, 31)))
    x = x.reshape(n, c, h + 2, 5, 32)
    x = jnp.transpose(x, (0, 2, 3, 1, 4)).reshape(n, h + 2, 5, 128)
    y = _conv_pool(x, conv1_w, conv1_b, P=32, B=4, nj=2, lane_cq=True)
    y = _conv_pool(y, conv2_w, conv2_b, P=4, B=4)
    y = _conv_pool(y, conv3_w, conv3_b, P=2, B=4, final=True)
    x = y.reshape(n, -1)
    return _fc_tail(x, fc1_w, fc1_b, fc2_w, fc2_b)


# speedup vs baseline: 1.6518x; 1.0102x over previous
import functools

import numpy as np

import jax
import jax.numpy as jnp
from jax.experimental import pallas as pl
from jax.experimental.pallas import tpu as pltpu


def _conv_compute(a, w_ref, b_ref, th, G):
    a = a.reshape(th * G, a.shape[-1])
    acc = jnp.dot(a, w_ref[...], preferred_element_type=jnp.float32)
    acc = jnp.maximum(acc + b_ref[...], 0.0)
    half = acc.shape[-1] // 2
    y = jnp.maximum(acc[:, :half], acc[:, half:])
    return y.reshape(th // 2, 2, G, half).max(axis=1)


def _store_packed(o_ref, i, y, th2):
    c2 = o_ref.shape[-1]
    g2 = o_ref.shape[-2] - 2
    y = y.reshape(th2, g2, c2).astype(o_ref.dtype)
    o_ref[i, 1:th2 + 1, 1:g2 + 1, :] = y
    zrow = jnp.zeros((1, g2 + 2, c2), o_ref.dtype)
    o_ref[i, 0:1] = zrow
    o_ref[i, th2 + 1:th2 + 2] = zrow
    zcol = jnp.zeros((th2 + 2, 1, c2), o_ref.dtype)
    o_ref[i, :, 0:1, :] = zcol
    o_ref[i, :, g2 + 1:g2 + 2, :] = zcol


def _conv_kernel(x_ref, w_ref, b_ref, o_ref, *, B, th, G, P, O, nj, final):
    for i in range(B):
        pieces = [x_ref[i, dh:dh + th, j:j + G, :]
                  for dh in (0, 1, 2) for j in range(nj)]
        a = jnp.concatenate(pieces, axis=-1)
        y = _conv_compute(a, w_ref, b_ref, th, G)
        th2 = th // 2
        if final:
            y = y.reshape(th2 * G, y.shape[-1]).astype(o_ref.dtype)
            o_ref[i] = jnp.transpose(y, (1, 0)).reshape(o_ref.shape[1:])
        else:
            _store_packed(o_ref, i, y, th2)


def _pack_w(w_oihw, P, lane_cq, nj):
    wt = jnp.transpose(w_oihw, (2, 3, 1, 0)).astype(jnp.bfloat16)
    order = list(range(0, P, 2)) + list(range(1, P, 2))
    sel = np.zeros((nj, P, P, 3), np.float32)
    for j in range(nj):
        for q in range(P):
            for slot, p in enumerate(order):
                dw = j * P + q - p if nj == 2 else (j - 1) * P + q - p + 1
                if 0 <= dw < 3:
                    sel[j, q, slot, dw] = 1.0
    out_order = 'hjcqpo' if lane_cq else 'hjqcpo'
    wp = jnp.einsum('jqpd,hdco->' + out_order,
                    jnp.asarray(sel, jnp.bfloat16), wt)
    rows = 3 * nj * P * w_oihw.shape[1]
    return wp.reshape(rows, P * w_oihw.shape[0])


_CPARAMS = pltpu.CompilerParams(
    dimension_semantics=("parallel",),
    vmem_limit_bytes=100 * 1024 * 1024)


def _conv_pool(xp, w_oihw, b, P, B, nj=3, final=False, lane_cq=False):
    n, h2, gtot, pc = xp.shape
    h, G = h2 - 2, gtot - (nj - 1)
    O = w_oihw.shape[0]
    wp = _pack_w(w_oihw, P, lane_cq=lane_cq, nj=nj)
    bp = jnp.tile(b.astype(jnp.float32), P).reshape(1, P * O)
    th2 = h // 2
    w2 = G * P // 2
    if final:
        out_sd = jax.ShapeDtypeStruct((n, O, th2 * w2), jnp.bfloat16)
        out_block = (B, O, th2 * w2)
    else:
        c2 = 128
        out_sd = jax.ShapeDtypeStruct((n, th2 + 2, w2 * O // c2 + 2, c2),
                                      jnp.bfloat16)
        out_block = (B, th2 + 2, w2 * O // c2 + 2, c2)
    body = functools.partial(_conv_kernel, B=B, th=h, G=G, P=P, O=O, nj=nj,
                             final=final)
    return pl.pallas_call(
        body,
        out_shape=out_sd,
        grid=(n // B,),
        in_specs=[
            pl.BlockSpec((B, h + 2, gtot, pc), lambda bi: (bi, 0, 0, 0)),
            pl.BlockSpec((3 * nj * pc, P * O), lambda bi: (0, 0)),
            pl.BlockSpec((1, P * O), lambda bi: (0, 0)),
        ],
        out_specs=pl.BlockSpec(out_block,
                               lambda bi: (bi,) + (0,) * (len(out_block) - 1)),
        compiler_params=_CPARAMS,
    )(xp, wp, bp)


def _fc1_kernel(x_ref, w1_ref, o_ref, acc_ref):
    k = pl.program_id(1)

    @pl.when(k == 0)
    def _init():
        acc_ref[...] = jnp.zeros_like(acc_ref)

    w = w1_ref[...].astype(jnp.bfloat16)
    acc_ref[...] += jax.lax.dot_general(
        x_ref[...], w, (((1,), (1,)), ((), ())),
        preferred_element_type=jnp.float32)

    @pl.when(k == pl.num_programs(1) - 1)
    def _finalize():
        o_ref[0] = acc_ref[...]


def _fc2_kernel(p_ref, b1_ref, w2_ref, b2_ref, o_ref):
    h = p_ref[0] + p_ref[1] + b1_ref[...]
    h = jnp.maximum(h, 0.0).astype(jnp.bfloat16)
    y = jax.lax.dot_general(
        h, w2_ref[...].astype(jnp.bfloat16), (((1,), (1,)), ((), ())),
        preferred_element_type=jnp.float32)
    o_ref[...] = y + b2_ref[...]


def _fc_tail(x, w1, b1, w2, b2, *, tk=8192):
    m, k = x.shape
    n1 = w1.shape[0]
    n2 = w2.shape[0]
    kt = k // tk
    part = pl.pallas_call(
        _fc1_kernel,
        out_shape=jax.ShapeDtypeStruct((2, m, n1), jnp.float32),
        grid=(2, kt // 2),
        in_specs=[
            pl.BlockSpec((m, tk), lambda c, kk: (0, c * (kt // 2) + kk)),
            pl.BlockSpec((n1, tk), lambda c, kk: (0, c * (kt // 2) + kk)),
        ],
        out_specs=pl.BlockSpec((1, m, n1), lambda c, kk: (c, 0, 0)),
        scratch_shapes=[pltpu.VMEM((m, n1), jnp.float32)],
        compiler_params=pltpu.CompilerParams(
            dimension_semantics=("parallel", "arbitrary"),
            vmem_limit_bytes=100 * 1024 * 1024),
    )(x, w1)
    b1p = b1.astype(jnp.float32).reshape(1, n1)
    b2p = b2.astype(jnp.float32).reshape(1, n2)
    return pl.pallas_call(
        _fc2_kernel,
        out_shape=jax.ShapeDtypeStruct((m, n2), jnp.float32),
        in_specs=[
            pl.BlockSpec((2, m, n1), lambda: (0, 0, 0)),
            pl.BlockSpec((1, n1), lambda: (0, 0)),
            pl.BlockSpec((n2, n1), lambda: (0, 0)),
            pl.BlockSpec((1, n2), lambda: (0, 0)),
        ],
        out_specs=pl.BlockSpec((m, n2), lambda: (0, 0)),
        compiler_params=pltpu.CompilerParams(
            vmem_limit_bytes=100 * 1024 * 1024),
    )(part, b1p, w2, b2p)


def kernel(conv1_w, conv1_b, conv2_w, conv2_b, conv3_w, conv3_b,
           fc1_w, fc1_b, fc2_w, fc2_b, x_nchw):
    n, c, h, w = x_nchw.shape
    x = jnp.pad(x_nchw.astype(jnp.bfloat16),
                ((0, 0), (0, 0), (1, 1), (1, 31)))
    x = x.reshape(n, c, h + 2, 5, 32)
    x = jnp.transpose(x, (0, 2, 3, 1, 4)).reshape(n, h + 2, 5, 128)
    y = _conv_pool(x, conv1_w, conv1_b, P=32, B=4, nj=2, lane_cq=True)
    y = _conv_pool(y, conv2_w, conv2_b, P=4, B=4)
    y = _conv_pool(y, conv3_w, conv3_b, P=2, B=4, final=True)
    x = y.reshape(n, -1)
    return _fc_tail(x, fc1_w, fc1_b, fc2_w, fc2_b)
